# Initial kernel scaffold; baseline (speedup 1.0000x reference)
import jax
import jax.numpy as jnp
from jax.experimental import pallas as pl


def kernel(x, edge_index, W1, b1, W2, b2, ln_scale, ln_bias):
    raise NotImplementedError("write your pallas kernel here")


# speedup vs baseline: 2.2747x; 2.2747x over previous
import dataclasses
import functools

import jax
import jax.numpy as jnp
from jax import lax
from jax.experimental import pallas as pl
from jax.experimental.pallas import tpu as pltpu
from jax.experimental.pallas import tpu_sc as plsc

N = 10000
E = 320000
D = 128
H = 64

NC = 2
NS = 16
NW = NC * NS

NPW = 320
NPAD = NW * NPW
TBL = 328
PAD_LOC = TBL - 1

EPW = E // NW
CB = 80
CD = 1280
GD = 32
MATBUF = CD + 64

_NEG = -3.0e38


def _sc_params():
    cp = pltpu.CompilerParams()
    if "needs_layout_passes" in pltpu.CompilerParams.__dataclass_fields__:
        cp = dataclasses.replace(cp, needs_layout_passes=False)
    return cp


def _stage_a_body(x_ref, w1a_ref, w1b_ref, b1_ref, t_ref):
    x = x_ref[...]
    xa = jnp.dot(x, w1a_ref[...], preferred_element_type=jnp.float32)
    xb = (
        jnp.dot(x, w1b_ref[...], preferred_element_type=jnp.float32)
        + b1_ref[...]
    )
    t_ref[...] = jnp.concatenate([xa, xb], axis=1)


def _stage_a(x, W1, b1):
    return pl.pallas_call(
        _stage_a_body,
        out_shape=jax.ShapeDtypeStruct((N, D), jnp.float32),
    )(x, W1[:D], W1[D:], b1.reshape(1, H))


def _stage_b_kernel(t_hbm, src_hbm, dst_hbm, h_hbm,
                    sidx, didx, abuf, bbuf, hbuf, sem_a, sem_b):
    c = lax.axis_index("c")
    s = lax.axis_index("s")
    wid = c * NS + s
    base = wid * EPW

    @pl.loop(0, EPW // CB)
    def _chunk(k):
        e0 = base + k * CB
        pltpu.sync_copy(src_hbm.at[pl.ds(e0, CB)], sidx)
        pltpu.sync_copy(dst_hbm.at[pl.ds(e0, CB)], didx)
        cpa = pltpu.async_copy(t_hbm.at[sidx], abuf, sem_a)
        cpb = pltpu.async_copy(t_hbm.at[didx], bbuf, sem_b)
        cpa.wait()
        cpb.wait()

        @pl.loop(0, CB)
        def _row(r):
            @pl.loop(0, H, step=16)
            def _col(t):
                hbuf[r, pl.ds(t, 16)] = jnp.maximum(
                    abuf[r, pl.ds(t, 16)] + bbuf[r, pl.ds(H + t, 16)], 0.0
                )

        pltpu.sync_copy(hbuf, h_hbm.at[pl.ds(e0, CB)])


def _stage_b(t, src, dst):
    mesh = plsc.VectorSubcoreMesh(core_axis_name="c", subcore_axis_name="s")
    kern = pl.kernel(
        _stage_b_kernel,
        out_type=jax.ShapeDtypeStruct((E, H), jnp.float32),
        mesh=mesh,
        scratch_types=[
            pltpu.VMEM((CB,), jnp.int32),
            pltpu.VMEM((CB,), jnp.int32),
            pltpu.VMEM((CB, D), jnp.float32),
            pltpu.VMEM((CB, D), jnp.float32),
            pltpu.VMEM((CB, H), jnp.float32),
            pltpu.SemaphoreType.DMA,
            pltpu.SemaphoreType.DMA,
        ],
        compiler_params=_sc_params(),
    )
    return kern(t, src, dst)


def _stage_c_body(h_ref, w2_ref, b2_ref, sc_ref, bi_ref, g_ref):
    y = jnp.dot(h_ref[...], w2_ref[...], preferred_element_type=jnp.float32)
    y = y + b2_ref[...]
    mu = jnp.mean(y, axis=-1, keepdims=True)
    var = jnp.mean((y - mu) ** 2, axis=-1, keepdims=True)
    g_ref[...] = (y - mu) * lax.rsqrt(var + 1e-5) * sc_ref[...] + bi_ref[...]


def _stage_c(h, W2, b2, ln_scale, ln_bias):
    BE = 3200
    grid = (E // BE,)
    return pl.pallas_call(
        _stage_c_body,
        grid=grid,
        in_specs=[
            pl.BlockSpec((BE, H), lambda i: (i, 0)),
            pl.BlockSpec((H, D), lambda i: (0, 0)),
            pl.BlockSpec((1, D), lambda i: (0, 0)),
            pl.BlockSpec((1, D), lambda i: (0, 0)),
            pl.BlockSpec((1, D), lambda i: (0, 0)),
        ],
        out_specs=pl.BlockSpec((BE, D), lambda i: (i, 0)),
        out_shape=jax.ShapeDtypeStruct((E, D), jnp.float32),
    )(h, W2, b2.reshape(1, D), ln_scale.reshape(1, D), ln_bias.reshape(1, D))


def _stage_d_kernel(g_hbm, src_hbm, out_hbm,
                    sv, mat, locv, rows, tsum, tmax,
                    cnt_smem, ptr_smem, sem_g):
    c = lax.axis_index("c")
    s = lax.axis_index("s")
    wid = c * NS + s
    lo = wid * NPW

    zero16 = jnp.zeros((16,), jnp.float32)
    neg16 = jnp.full((16,), _NEG, jnp.float32)

    @pl.loop(0, TBL)
    def _init(r):
        @pl.loop(0, D, step=16)
        def _initc(t):
            tsum[r, pl.ds(t, 16)] = zero16
            tmax[r, pl.ds(t, 16)] = neg16

    @pl.loop(0, TBL)
    def _initcnt(r):
        cnt_smem[r] = 0

    ptr_smem[0] = 0
    lane = lax.iota(jnp.int32, 16)
    true16 = lane < 16

    def _process_groups(ngroups):
        @pl.loop(0, ngroups)
        def _grp(k):
            pltpu.async_copy(
                g_hbm.at[mat.at[pl.ds(k * GD, GD)]], rows, sem_g
            ).wait()

            @pl.loop(0, GD)
            def _edge(i):
                lv = locv[pl.ds(k * GD + (i // 16) * 16, 16)]
                l = jnp.max(jnp.where(lane == i % 16, lv, -1))
                cnt_smem[l] = cnt_smem[l] + 1

                @pl.loop(0, D, step=16)
                def _feat(t):
                    v = rows[i, pl.ds(t, 16)]
                    slc = (l, pl.ds(t, 16))
                    tsum[slc] = tsum[slc] + v
                    tmax[slc] = jnp.maximum(tmax[slc], v)

    @pl.loop(0, E // CD)
    def _chunk(kc):
        pltpu.sync_copy(src_hbm.at[pl.ds(kc * CD, CD)], sv)

        @pl.loop(0, CD // 16)
        def _vec(j):
            sq = sv[pl.ds(j * 16, 16)]
            m = (sq >= lo) & (sq < lo + NPW)
            eid = kc * CD + j * 16 + lane
            cnt = ptr_smem[0]
            plsc.store_compressed(mat.at[pl.ds(cnt, 16)], eid, mask=m)
            plsc.store_compressed(locv.at[pl.ds(cnt, 16)], sq - lo, mask=m)
            ptr_smem[0] = cnt + jnp.sum(m.astype(jnp.int32))

        cnt = ptr_smem[0]
        g0 = cnt // GD
        _process_groups(g0)
        rem0 = g0 * GD
        mat[pl.ds(0, 16)] = mat[pl.ds(rem0, 16)]
        mat[pl.ds(16, 16)] = mat[pl.ds(rem0 + 16, 16)]
        locv[pl.ds(0, 16)] = locv[pl.ds(rem0, 16)]
        locv[pl.ds(16, 16)] = locv[pl.ds(rem0 + 16, 16)]
        ptr_smem[0] = cnt - rem0

    cnt = ptr_smem[0]
    plsc.store_compressed(mat.at[pl.ds(cnt, 16)],
                          jnp.zeros((16,), jnp.int32), mask=true16)
    plsc.store_compressed(mat.at[pl.ds(cnt + 16, 16)],
                          jnp.zeros((16,), jnp.int32), mask=true16)
    plsc.store_compressed(locv.at[pl.ds(cnt, 16)],
                          jnp.full((16,), PAD_LOC, jnp.int32), mask=true16)
    plsc.store_compressed(locv.at[pl.ds(cnt + 16, 16)],
                          jnp.full((16,), PAD_LOC, jnp.int32), mask=true16)
    _process_groups((cnt + GD - 1) // GD)

    @pl.loop(0, NPW)
    def _node(r):
        ci = cnt_smem[r]
        cv = jnp.broadcast_to(ci, (16,))
        cf = cv.astype(jnp.float32)
        inv = 1.0 / jnp.maximum(cf, 1.0)
        nonz = cv > 0

        @pl.loop(0, D, step=16)
        def _feat(t):
            slc = (r, pl.ds(t, 16))
            mean = tsum[slc] * inv
            mx = jnp.where(nonz, tmax[slc], 0.0)
            tsum[slc] = mean + mx

    pltpu.sync_copy(tsum.at[pl.ds(0, NPW)], out_hbm.at[pl.ds(lo, NPW)])


def _stage_d(g, src):
    mesh = plsc.VectorSubcoreMesh(core_axis_name="c", subcore_axis_name="s")
    kern = pl.kernel(
        _stage_d_kernel,
        out_type=jax.ShapeDtypeStruct((NPAD, D), jnp.float32),
        mesh=mesh,
        scratch_types=[
            pltpu.VMEM((CD,), jnp.int32),
            pltpu.VMEM((MATBUF,), jnp.int32),
            pltpu.VMEM((MATBUF,), jnp.int32),
            pltpu.VMEM((GD, D), jnp.float32),
            pltpu.VMEM((TBL, D), jnp.float32),
            pltpu.VMEM((TBL, D), jnp.float32),
            pltpu.SMEM((TBL,), jnp.int32),
            pltpu.SMEM((1,), jnp.int32),
            pltpu.SemaphoreType.DMA,
        ],
        compiler_params=_sc_params(),
    )
    return kern(g, src)


@functools.partial(jax.jit)
def kernel(x, edge_index, W1, b1, W2, b2, ln_scale, ln_bias):
    src = edge_index[0].astype(jnp.int32)
    dst = edge_index[1].astype(jnp.int32)
    t = _stage_a(x, W1, b1)
    h = _stage_b(t, src, dst)
    g = _stage_c(h, W2, b2, ln_scale, ln_bias)
    out = _stage_d(g, src)
    return out[:N]


# speedup vs baseline: 2.6872x; 1.1813x over previous
import dataclasses
import functools

import jax
import jax.numpy as jnp
from jax import lax
from jax.experimental import pallas as pl
from jax.experimental.pallas import tpu as pltpu
from jax.experimental.pallas import tpu_sc as plsc

N = 10000
E = 320000
D = 128
H = 64

NC = 2
NS = 16
NW = NC * NS

NPW = 320
NPAD = NW * NPW
TBL = 328
PAD_LOC = TBL - 1

EPW = E // NW
CB = 80
CD = 1280
GD = 32
MATBUF = 1600

_NEG = -3.0e38


def _sc_params():
    cp = pltpu.CompilerParams()
    if "needs_layout_passes" in pltpu.CompilerParams.__dataclass_fields__:
        cp = dataclasses.replace(cp, needs_layout_passes=False)
    return cp


def _stage_a_body(x_ref, w1a_ref, w1b_ref, b1_ref, t_ref):
    x = x_ref[...]
    xa = jnp.dot(x, w1a_ref[...], preferred_element_type=jnp.float32)
    xb = (
        jnp.dot(x, w1b_ref[...], preferred_element_type=jnp.float32)
        + b1_ref[...]
    )
    t_ref[...] = jnp.concatenate([xa, xb], axis=1)


def _stage_a(x, W1, b1):
    return pl.pallas_call(
        _stage_a_body,
        out_shape=jax.ShapeDtypeStruct((N, D), jnp.float32),
    )(x, W1[:D], W1[D:], b1.reshape(1, H))


def _stage_b_kernel(t_hbm, src_hbm, dst_hbm, h_hbm,
                    sidx, didx, abuf, bbuf, hbuf, sem_a, sem_b):
    c = lax.axis_index("c")
    s = lax.axis_index("s")
    wid = c * NS + s
    base = wid * EPW

    @pl.loop(0, EPW // CB)
    def _chunk(k):
        e0 = base + k * CB
        pltpu.sync_copy(src_hbm.at[pl.ds(e0, CB)], sidx)
        pltpu.sync_copy(dst_hbm.at[pl.ds(e0, CB)], didx)
        cpa = pltpu.async_copy(t_hbm.at[sidx], abuf, sem_a)
        cpb = pltpu.async_copy(t_hbm.at[didx], bbuf, sem_b)
        cpa.wait()
        cpb.wait()

        @pl.loop(0, CB)
        def _row(r):
            @pl.loop(0, H, step=16)
            def _col(t):
                hbuf[r, pl.ds(t, 16)] = jnp.maximum(
                    abuf[r, pl.ds(t, 16)] + bbuf[r, pl.ds(H + t, 16)], 0.0
                )

        pltpu.sync_copy(hbuf, h_hbm.at[pl.ds(e0, CB)])


def _stage_b(t, src, dst):
    mesh = plsc.VectorSubcoreMesh(core_axis_name="c", subcore_axis_name="s")
    kern = pl.kernel(
        _stage_b_kernel,
        out_type=jax.ShapeDtypeStruct((E, H), jnp.float32),
        mesh=mesh,
        scratch_types=[
            pltpu.VMEM((CB,), jnp.int32),
            pltpu.VMEM((CB,), jnp.int32),
            pltpu.VMEM((CB, D), jnp.float32),
            pltpu.VMEM((CB, D), jnp.float32),
            pltpu.VMEM((CB, H), jnp.float32),
            pltpu.SemaphoreType.DMA,
            pltpu.SemaphoreType.DMA,
        ],
        compiler_params=_sc_params(),
    )
    return kern(t, src, dst)


def _stage_c_body(h_ref, w2_ref, b2_ref, sc_ref, bi_ref, g_ref):
    y = jnp.dot(h_ref[...], w2_ref[...], preferred_element_type=jnp.float32)
    y = y + b2_ref[...]
    mu = jnp.mean(y, axis=-1, keepdims=True)
    var = jnp.mean((y - mu) ** 2, axis=-1, keepdims=True)
    g_ref[...] = (y - mu) * lax.rsqrt(var + 1e-5) * sc_ref[...] + bi_ref[...]


def _stage_c(h, W2, b2, ln_scale, ln_bias):
    BE = 3200
    grid = (E // BE,)
    return pl.pallas_call(
        _stage_c_body,
        grid=grid,
        in_specs=[
            pl.BlockSpec((BE, H), lambda i: (i, 0)),
            pl.BlockSpec((H, D), lambda i: (0, 0)),
            pl.BlockSpec((1, D), lambda i: (0, 0)),
            pl.BlockSpec((1, D), lambda i: (0, 0)),
            pl.BlockSpec((1, D), lambda i: (0, 0)),
        ],
        out_specs=pl.BlockSpec((BE, D), lambda i: (i, 0)),
        out_shape=jax.ShapeDtypeStruct((E, D), jnp.float32),
    )(h, W2, b2.reshape(1, D), ln_scale.reshape(1, D), ln_bias.reshape(1, D))


DRAIN = 256


def _stage_d_kernel(g_hbm, src_hbm, out_hbm,
                    sv, mat, locv, rows0, rows1,
                    tsum, tmax,
                    cnt_smem, ptr_smem, sem0, sem1):
    c = lax.axis_index("c")
    s = lax.axis_index("s")
    wid = c * NS + s
    lo = wid * NPW

    zero16 = jnp.zeros((16,), jnp.float32)
    neg16 = jnp.full((16,), _NEG, jnp.float32)
    lane = lax.iota(jnp.int32, 16)
    true16 = lane < 16

    @pl.loop(0, TBL)
    def _init(r):
        @pl.loop(0, D, step=16)
        def _initc(t):
            tsum[r, pl.ds(t, 16)] = zero16

    @pl.loop(0, TBL)
    def _initcnt(r):
        cnt_smem[r] = 0

    @pl.loop(0, TBL)
    def _initm(r):
        @pl.loop(0, D, step=16)
        def _initmc(t):
            tmax[r, pl.ds(t, 16)] = neg16

    ptr_smem[0] = 0

    def _gather(k, rows, sem):
        return pltpu.async_copy(
            g_hbm.at[mat.at[pl.ds(k * GD, GD)]], rows, sem)

    def _consume(k, rows):
        @pl.loop(0, GD)
        def _edge(i):
            lv = locv[pl.ds(k * GD + (i // 16) * 16, 16)]
            l = jnp.max(jnp.where(lane == i % 16, lv, -1))
            cnt_smem[l] = cnt_smem[l] + 1

            @pl.loop(0, D, step=16)
            def _feat(t):
                v = rows[i, pl.ds(t, 16)]
                plsc.addupdate(tsum.at[l, pl.ds(t, 16)], v)
                tmax[l, pl.ds(t, 16)] = jnp.maximum(tmax[l, pl.ds(t, 16)], v)

    def _process_groups(ngroups):
        @pl.when(ngroups > 0)
        def _():
            _gather(0, rows0, sem0)

        @pl.loop(0, ngroups)
        def _grp(k):
            @pl.when(k % 2 == 0)
            def _():
                pltpu.make_async_copy(
                    g_hbm.at[mat.at[pl.ds(k * GD, GD)]], rows0, sem0).wait()

                @pl.when(k + 1 < ngroups)
                def _():
                    _gather(k + 1, rows1, sem1)
                _consume(k, rows0)

            @pl.when(k % 2 == 1)
            def _():
                pltpu.make_async_copy(
                    g_hbm.at[mat.at[pl.ds(k * GD, GD)]], rows1, sem1).wait()

                @pl.when(k + 1 < ngroups)
                def _():
                    _gather(k + 1, rows0, sem0)
                _consume(k, rows1)

    def _drain():
        cnt = ptr_smem[0]
        g0 = cnt // GD
        _process_groups(g0)
        rem0 = g0 * GD
        mat[pl.ds(0, 16)] = mat[pl.ds(rem0, 16)]
        mat[pl.ds(16, 16)] = mat[pl.ds(rem0 + 16, 16)]
        locv[pl.ds(0, 16)] = locv[pl.ds(rem0, 16)]
        locv[pl.ds(16, 16)] = locv[pl.ds(rem0 + 16, 16)]
        ptr_smem[0] = cnt - rem0

    @pl.loop(0, E // CD)
    def _chunk(kc):
        pltpu.sync_copy(src_hbm.at[pl.ds(kc * CD, CD)], sv)

        @pl.loop(0, CD // 16)
        def _vec(j):
            sq = sv[pl.ds(j * 16, 16)]
            m = (sq >= lo) & (sq < lo + NPW)
            eid = kc * CD + j * 16 + lane
            cnt = ptr_smem[0]
            plsc.store_compressed(mat.at[pl.ds(cnt, 16)], eid, mask=m)
            plsc.store_compressed(locv.at[pl.ds(cnt, 16)], sq - lo, mask=m)
            ptr_smem[0] = cnt + jnp.sum(m.astype(jnp.int32))

        @pl.when(ptr_smem[0] >= DRAIN)
        def _():
            _drain()

    cnt = ptr_smem[0]
    plsc.store_compressed(mat.at[pl.ds(cnt, 16)],
                          jnp.zeros((16,), jnp.int32), mask=true16)
    plsc.store_compressed(mat.at[pl.ds(cnt + 16, 16)],
                          jnp.zeros((16,), jnp.int32), mask=true16)
    plsc.store_compressed(locv.at[pl.ds(cnt, 16)],
                          jnp.full((16,), PAD_LOC, jnp.int32), mask=true16)
    plsc.store_compressed(locv.at[pl.ds(cnt + 16, 16)],
                          jnp.full((16,), PAD_LOC, jnp.int32), mask=true16)
    _process_groups((cnt + GD - 1) // GD)

    @pl.loop(0, NPW)
    def _node(r):
        ci = cnt_smem[r]
        cf = jnp.broadcast_to(ci, (16,)).astype(jnp.float32)
        inv = 1.0 / jnp.maximum(cf, 1.0)
        nz = cf > 0.0

        @pl.loop(0, D, step=16)
        def _feat(t):
            mean = tsum[r, pl.ds(t, 16)] * inv
            mx = jnp.where(nz, tmax[r, pl.ds(t, 16)], 0.0)
            tsum[r, pl.ds(t, 16)] = mean + mx

    pltpu.sync_copy(tsum.at[pl.ds(0, NPW)], out_hbm.at[pl.ds(lo, NPW)])


def _stage_d(g, src):
    mesh = plsc.VectorSubcoreMesh(core_axis_name="c", subcore_axis_name="s")
    kern = pl.kernel(
        _stage_d_kernel,
        out_type=jax.ShapeDtypeStruct((NPAD, D), jnp.float32),
        mesh=mesh,
        scratch_types=[
            pltpu.VMEM((CD,), jnp.int32),
            pltpu.VMEM((MATBUF,), jnp.int32),
            pltpu.VMEM((MATBUF,), jnp.int32),
            pltpu.VMEM((GD, D), jnp.float32),
            pltpu.VMEM((GD, D), jnp.float32),
            pltpu.VMEM((TBL, D), jnp.float32),
            pltpu.VMEM((TBL, D), jnp.float32),
            pltpu.SMEM((TBL,), jnp.int32),
            pltpu.SMEM((1,), jnp.int32),
            pltpu.SemaphoreType.DMA,
            pltpu.SemaphoreType.DMA,
        ],
        compiler_params=_sc_params(),
    )
    return kern(g, src)


@functools.partial(jax.jit)
def kernel(x, edge_index, W1, b1, W2, b2, ln_scale, ln_bias):
    src = edge_index[0].astype(jnp.int32)
    dst = edge_index[1].astype(jnp.int32)
    t = _stage_a(x, W1, b1)
    h = _stage_b(t, src, dst)
    g = _stage_c(h, W2, b2, ln_scale, ln_bias)
    out = _stage_d(g, src)
    return out[:N]


# speedup vs baseline: 3.0676x; 1.1416x over previous
import dataclasses
import functools

import jax
import jax.numpy as jnp
from jax import lax
from jax.experimental import pallas as pl
from jax.experimental.pallas import tpu as pltpu
from jax.experimental.pallas import tpu_sc as plsc

N = 10000
E = 320000
D = 128
H = 64

NC = 2
NS = 16
NW = NC * NS

NPW = 320
NPAD = NW * NPW
TBL = 328
PAD_LOC = TBL - 1

EPW = E // NW
CB = 80
CD = 1280
GD = 32
MATBUF = 1600

_NEG = -3.0e38


def _sc_params():
    cp = pltpu.CompilerParams()
    if "needs_layout_passes" in pltpu.CompilerParams.__dataclass_fields__:
        cp = dataclasses.replace(cp, needs_layout_passes=False)
    return cp


def _stage_a_body(x_ref, w1a_ref, w1b_ref, b1_ref, t_ref):
    x = x_ref[...]
    xa = jnp.dot(x, w1a_ref[...], preferred_element_type=jnp.float32)
    xb = (
        jnp.dot(x, w1b_ref[...], preferred_element_type=jnp.float32)
        + b1_ref[...]
    )
    t_ref[...] = jnp.concatenate([xa, xb], axis=1)


def _stage_a(x, W1, b1):
    return pl.pallas_call(
        _stage_a_body,
        out_shape=jax.ShapeDtypeStruct((N, D), jnp.float32),
    )(x, W1[:D], W1[D:], b1.reshape(1, H))


def _stage_b_kernel(t_hbm, src_hbm, dst_hbm, h_hbm,
                    sidx, didx, abuf0, bbuf0, hbuf0, abuf1, bbuf1, hbuf1,
                    sem_a0, sem_b0, sem_a1, sem_b1, sem_w0, sem_w1):
    c = lax.axis_index("c")
    s = lax.axis_index("s")
    wid = c * NS + s
    base = wid * EPW
    NCH = EPW // CB

    pltpu.sync_copy(src_hbm.at[pl.ds(base, EPW)], sidx)
    pltpu.sync_copy(dst_hbm.at[pl.ds(base, EPW)], didx)

    def _issue(k, abuf, bbuf, sa, sb):
        pltpu.async_copy(t_hbm.at[sidx.at[pl.ds(k * CB, CB)]], abuf, sa)
        pltpu.async_copy(t_hbm.at[didx.at[pl.ds(k * CB, CB)]], bbuf, sb)

    def _wait(k, abuf, bbuf, sa, sb):
        pltpu.make_async_copy(t_hbm.at[sidx.at[pl.ds(k * CB, CB)]], abuf, sa).wait()
        pltpu.make_async_copy(t_hbm.at[didx.at[pl.ds(k * CB, CB)]], bbuf, sb).wait()

    def _phase(k, abuf, bbuf, hbuf, sa, sb, sw, oab, obb, osa, osb):
        _wait(k, abuf, bbuf, sa, sb)

        @pl.when(k + 1 < NCH)
        def _():
            _issue(k + 1, oab, obb, osa, osb)

        @pl.when(k >= 2)
        def _():
            pltpu.make_async_copy(hbuf, h_hbm.at[pl.ds(base, CB)], sw).wait()

        @pl.loop(0, CB)
        def _row(r):
            @pl.loop(0, H, step=16)
            def _col(t):
                hbuf[r, pl.ds(t, 16)] = jnp.maximum(
                    abuf[r, pl.ds(t, 16)] + bbuf[r, pl.ds(H + t, 16)], 0.0
                )

        pltpu.async_copy(hbuf, h_hbm.at[pl.ds(base + k * CB, CB)], sw)

    _issue(0, abuf0, bbuf0, sem_a0, sem_b0)

    @pl.loop(0, NCH)
    def _chunk(k):
        @pl.when(k % 2 == 0)
        def _():
            _phase(k, abuf0, bbuf0, hbuf0, sem_a0, sem_b0, sem_w0,
                   abuf1, bbuf1, sem_a1, sem_b1)

        @pl.when(k % 2 == 1)
        def _():
            _phase(k, abuf1, bbuf1, hbuf1, sem_a1, sem_b1, sem_w1,
                   abuf0, bbuf0, sem_a0, sem_b0)

    pltpu.make_async_copy(hbuf0, h_hbm.at[pl.ds(base, CB)], sem_w0).wait()
    pltpu.make_async_copy(hbuf1, h_hbm.at[pl.ds(base, CB)], sem_w1).wait()


def _stage_b(t, src, dst):
    mesh = plsc.VectorSubcoreMesh(core_axis_name="c", subcore_axis_name="s")
    kern = pl.kernel(
        _stage_b_kernel,
        out_type=jax.ShapeDtypeStruct((E, H), jnp.float32),
        mesh=mesh,
        scratch_types=[
            pltpu.VMEM((EPW,), jnp.int32),
            pltpu.VMEM((EPW,), jnp.int32),
            pltpu.VMEM((CB, D), jnp.float32),
            pltpu.VMEM((CB, D), jnp.float32),
            pltpu.VMEM((CB, H), jnp.float32),
            pltpu.VMEM((CB, D), jnp.float32),
            pltpu.VMEM((CB, D), jnp.float32),
            pltpu.VMEM((CB, H), jnp.float32),
            pltpu.SemaphoreType.DMA,
            pltpu.SemaphoreType.DMA,
            pltpu.SemaphoreType.DMA,
            pltpu.SemaphoreType.DMA,
            pltpu.SemaphoreType.DMA,
            pltpu.SemaphoreType.DMA,
        ],
        compiler_params=_sc_params(),
    )
    return kern(t, src, dst)


def _stage_c_body(h_ref, w2_ref, b2_ref, sc_ref, bi_ref, g_ref):
    y = jnp.dot(h_ref[...], w2_ref[...], preferred_element_type=jnp.float32)
    y = y + b2_ref[...]
    mu = jnp.mean(y, axis=-1, keepdims=True)
    var = jnp.mean((y - mu) ** 2, axis=-1, keepdims=True)
    g_ref[...] = (y - mu) * lax.rsqrt(var + 1e-5) * sc_ref[...] + bi_ref[...]


def _stage_c(h, W2, b2, ln_scale, ln_bias):
    BE = 3200
    grid = (E // BE,)
    return pl.pallas_call(
        _stage_c_body,
        grid=grid,
        in_specs=[
            pl.BlockSpec((BE, H), lambda i: (i, 0)),
            pl.BlockSpec((H, D), lambda i: (0, 0)),
            pl.BlockSpec((1, D), lambda i: (0, 0)),
            pl.BlockSpec((1, D), lambda i: (0, 0)),
            pl.BlockSpec((1, D), lambda i: (0, 0)),
        ],
        out_specs=pl.BlockSpec((BE, D), lambda i: (i, 0)),
        out_shape=jax.ShapeDtypeStruct((E, D), jnp.float32),
    )(h, W2, b2.reshape(1, D), ln_scale.reshape(1, D), ln_bias.reshape(1, D))


DRAIN = 256


def _stage_d_kernel(g_hbm, src_hbm, out_hbm,
                    sv, mat, locv, rows0, rows1,
                    tsum, tmax,
                    cnt_smem, ptr_smem, sem0, sem1):
    c = lax.axis_index("c")
    s = lax.axis_index("s")
    wid = c * NS + s
    lo = wid * NPW

    zero16 = jnp.zeros((16,), jnp.float32)
    neg16 = jnp.full((16,), _NEG, jnp.float32)
    lane = lax.iota(jnp.int32, 16)
    true16 = lane < 16

    @pl.loop(0, TBL)
    def _init(r):
        @pl.loop(0, D, step=16)
        def _initc(t):
            tsum[r, pl.ds(t, 16)] = zero16

    @pl.loop(0, TBL)
    def _initcnt(r):
        cnt_smem[r] = 0

    @pl.loop(0, TBL)
    def _initm(r):
        @pl.loop(0, D, step=16)
        def _initmc(t):
            tmax[r, pl.ds(t, 16)] = neg16

    ptr_smem[0] = 0

    def _gather(k, rows, sem):
        return pltpu.async_copy(
            g_hbm.at[mat.at[pl.ds(k * GD, GD)]], rows, sem)

    def _consume(k, rows):
        @pl.loop(0, GD)
        def _edge(i):
            lv = locv[pl.ds(k * GD + (i // 16) * 16, 16)]
            l = jnp.max(jnp.where(lane == i % 16, lv, -1))
            cnt_smem[l] = cnt_smem[l] + 1

            @pl.loop(0, D, step=16)
            def _feat(t):
                v = rows[i, pl.ds(t, 16)]
                plsc.addupdate(tsum.at[l, pl.ds(t, 16)], v)
                tmax[l, pl.ds(t, 16)] = jnp.maximum(tmax[l, pl.ds(t, 16)], v)

    def _process_groups(ngroups):
        @pl.when(ngroups > 0)
        def _():
            _gather(0, rows0, sem0)

        @pl.loop(0, ngroups)
        def _grp(k):
            @pl.when(k % 2 == 0)
            def _():
                pltpu.make_async_copy(
                    g_hbm.at[mat.at[pl.ds(k * GD, GD)]], rows0, sem0).wait()

                @pl.when(k + 1 < ngroups)
                def _():
                    _gather(k + 1, rows1, sem1)
                _consume(k, rows0)

            @pl.when(k % 2 == 1)
            def _():
                pltpu.make_async_copy(
                    g_hbm.at[mat.at[pl.ds(k * GD, GD)]], rows1, sem1).wait()

                @pl.when(k + 1 < ngroups)
                def _():
                    _gather(k + 1, rows0, sem0)
                _consume(k, rows1)

    def _drain():
        cnt = ptr_smem[0]
        g0 = cnt // GD
        _process_groups(g0)
        rem0 = g0 * GD
        mat[pl.ds(0, 16)] = mat[pl.ds(rem0, 16)]
        mat[pl.ds(16, 16)] = mat[pl.ds(rem0 + 16, 16)]
        locv[pl.ds(0, 16)] = locv[pl.ds(rem0, 16)]
        locv[pl.ds(16, 16)] = locv[pl.ds(rem0 + 16, 16)]
        ptr_smem[0] = cnt - rem0

    @pl.loop(0, E // CD)
    def _chunk(kc):
        pltpu.sync_copy(src_hbm.at[pl.ds(kc * CD, CD)], sv)

        @pl.loop(0, CD // 16)
        def _vec(j):
            sq = sv[pl.ds(j * 16, 16)]
            m = (sq >= lo) & (sq < lo + NPW)
            eid = kc * CD + j * 16 + lane
            cnt = ptr_smem[0]
            plsc.store_compressed(mat.at[pl.ds(cnt, 16)], eid, mask=m)
            plsc.store_compressed(locv.at[pl.ds(cnt, 16)], sq - lo, mask=m)
            ptr_smem[0] = cnt + jnp.sum(m.astype(jnp.int32))

        @pl.when(ptr_smem[0] >= DRAIN)
        def _():
            _drain()

    cnt = ptr_smem[0]
    plsc.store_compressed(mat.at[pl.ds(cnt, 16)],
                          jnp.zeros((16,), jnp.int32), mask=true16)
    plsc.store_compressed(mat.at[pl.ds(cnt + 16, 16)],
                          jnp.zeros((16,), jnp.int32), mask=true16)
    plsc.store_compressed(locv.at[pl.ds(cnt, 16)],
                          jnp.full((16,), PAD_LOC, jnp.int32), mask=true16)
    plsc.store_compressed(locv.at[pl.ds(cnt + 16, 16)],
                          jnp.full((16,), PAD_LOC, jnp.int32), mask=true16)
    _process_groups((cnt + GD - 1) // GD)

    @pl.loop(0, NPW)
    def _node(r):
        ci = cnt_smem[r]
        cf = jnp.broadcast_to(ci, (16,)).astype(jnp.float32)
        inv = 1.0 / jnp.maximum(cf, 1.0)
        nz = cf > 0.0

        @pl.loop(0, D, step=16)
        def _feat(t):
            mean = tsum[r, pl.ds(t, 16)] * inv
            mx = jnp.where(nz, tmax[r, pl.ds(t, 16)], 0.0)
            tsum[r, pl.ds(t, 16)] = mean + mx

    pltpu.sync_copy(tsum.at[pl.ds(0, NPW)], out_hbm.at[pl.ds(lo, NPW)])


def _stage_d(g, src):
    mesh = plsc.VectorSubcoreMesh(core_axis_name="c", subcore_axis_name="s")
    kern = pl.kernel(
        _stage_d_kernel,
        out_type=jax.ShapeDtypeStruct((NPAD, D), jnp.float32),
        mesh=mesh,
        scratch_types=[
            pltpu.VMEM((CD,), jnp.int32),
            pltpu.VMEM((MATBUF,), jnp.int32),
            pltpu.VMEM((MATBUF,), jnp.int32),
            pltpu.VMEM((GD, D), jnp.float32),
            pltpu.VMEM((GD, D), jnp.float32),
            pltpu.VMEM((TBL, D), jnp.float32),
            pltpu.VMEM((TBL, D), jnp.float32),
            pltpu.SMEM((TBL,), jnp.int32),
            pltpu.SMEM((1,), jnp.int32),
            pltpu.SemaphoreType.DMA,
            pltpu.SemaphoreType.DMA,
        ],
        compiler_params=_sc_params(),
    )
    return kern(g, src)


@functools.partial(jax.jit)
def kernel(x, edge_index, W1, b1, W2, b2, ln_scale, ln_bias):
    src = edge_index[0].astype(jnp.int32)
    dst = edge_index[1].astype(jnp.int32)
    t = _stage_a(x, W1, b1)
    h = _stage_b(t, src, dst)
    g = _stage_c(h, W2, b2, ln_scale, ln_bias)
    out = _stage_d(g, src)
    return out[:N]


# speedup vs baseline: 3.3592x; 1.0950x over previous
import dataclasses
import functools

import jax
import jax.numpy as jnp
from jax import lax
from jax.experimental import pallas as pl
from jax.experimental.pallas import tpu as pltpu
from jax.experimental.pallas import tpu_sc as plsc

N = 10000
E = 320000
D = 128
H = 64

NC = 2
NS = 16
NW = NC * NS

NPW = 320
NPAD = NW * NPW
TBL = 328
PAD_LOC = TBL - 1

EPW = E // NW
CB = 80
CD = 1280
GD = 64
MATBUF = 1600

_NEG = -3.0e38


def _sc_params():
    cp = pltpu.CompilerParams()
    if "needs_layout_passes" in pltpu.CompilerParams.__dataclass_fields__:
        cp = dataclasses.replace(cp, needs_layout_passes=False)
    return cp


def _stage_a_body(x_ref, w1a_ref, w1b_ref, b1_ref, t_ref):
    x = x_ref[...]
    xa = jnp.dot(x, w1a_ref[...], preferred_element_type=jnp.float32)
    xb = (
        jnp.dot(x, w1b_ref[...], preferred_element_type=jnp.float32)
        + b1_ref[...]
    )
    t_ref[...] = jnp.concatenate([xa, xb], axis=1)


def _stage_a(x, W1, b1):
    return pl.pallas_call(
        _stage_a_body,
        out_shape=jax.ShapeDtypeStruct((N, D), jnp.float32),
    )(x, W1[:D], W1[D:], b1.reshape(1, H))


def _stage_b_kernel(t_hbm, src_hbm, dst_hbm, h_hbm,
                    sidx, didx, abuf0, bbuf0, hbuf0, abuf1, bbuf1, hbuf1,
                    sem_a0, sem_b0, sem_a1, sem_b1, sem_w0, sem_w1):
    c = lax.axis_index("c")
    s = lax.axis_index("s")
    wid = c * NS + s
    base = wid * EPW
    NCH = EPW // CB

    pltpu.sync_copy(src_hbm.at[pl.ds(base, EPW)], sidx)
    pltpu.sync_copy(dst_hbm.at[pl.ds(base, EPW)], didx)

    def _issue(k, abuf, bbuf, sa, sb):
        pltpu.async_copy(t_hbm.at[sidx.at[pl.ds(k * CB, CB)]], abuf, sa)
        pltpu.async_copy(t_hbm.at[didx.at[pl.ds(k * CB, CB)]], bbuf, sb)

    def _wait(k, abuf, bbuf, sa, sb):
        pltpu.make_async_copy(t_hbm.at[sidx.at[pl.ds(k * CB, CB)]], abuf, sa).wait()
        pltpu.make_async_copy(t_hbm.at[didx.at[pl.ds(k * CB, CB)]], bbuf, sb).wait()

    def _phase(k, abuf, bbuf, hbuf, sa, sb, sw, oab, obb, osa, osb):
        _wait(k, abuf, bbuf, sa, sb)

        @pl.when(k + 1 < NCH)
        def _():
            _issue(k + 1, oab, obb, osa, osb)

        @pl.when(k >= 2)
        def _():
            pltpu.make_async_copy(hbuf, h_hbm.at[pl.ds(base, CB)], sw).wait()

        @pl.loop(0, CB)
        def _row(r):
            @pl.loop(0, H, step=16)
            def _col(t):
                hbuf[r, pl.ds(t, 16)] = jnp.maximum(
                    abuf[r, pl.ds(t, 16)] + bbuf[r, pl.ds(H + t, 16)], 0.0
                )

        pltpu.async_copy(hbuf, h_hbm.at[pl.ds(base + k * CB, CB)], sw)

    _issue(0, abuf0, bbuf0, sem_a0, sem_b0)

    @pl.loop(0, NCH)
    def _chunk(k):
        @pl.when(k % 2 == 0)
        def _():
            _phase(k, abuf0, bbuf0, hbuf0, sem_a0, sem_b0, sem_w0,
                   abuf1, bbuf1, sem_a1, sem_b1)

        @pl.when(k % 2 == 1)
        def _():
            _phase(k, abuf1, bbuf1, hbuf1, sem_a1, sem_b1, sem_w1,
                   abuf0, bbuf0, sem_a0, sem_b0)

    pltpu.make_async_copy(hbuf0, h_hbm.at[pl.ds(base, CB)], sem_w0).wait()
    pltpu.make_async_copy(hbuf1, h_hbm.at[pl.ds(base, CB)], sem_w1).wait()


def _stage_b(t, src, dst):
    mesh = plsc.VectorSubcoreMesh(core_axis_name="c", subcore_axis_name="s")
    kern = pl.kernel(
        _stage_b_kernel,
        out_type=jax.ShapeDtypeStruct((E, H), jnp.float32),
        mesh=mesh,
        scratch_types=[
            pltpu.VMEM((EPW,), jnp.int32),
            pltpu.VMEM((EPW,), jnp.int32),
            pltpu.VMEM((CB, D), jnp.float32),
            pltpu.VMEM((CB, D), jnp.float32),
            pltpu.VMEM((CB, H), jnp.float32),
            pltpu.VMEM((CB, D), jnp.float32),
            pltpu.VMEM((CB, D), jnp.float32),
            pltpu.VMEM((CB, H), jnp.float32),
            pltpu.SemaphoreType.DMA,
            pltpu.SemaphoreType.DMA,
            pltpu.SemaphoreType.DMA,
            pltpu.SemaphoreType.DMA,
            pltpu.SemaphoreType.DMA,
            pltpu.SemaphoreType.DMA,
        ],
        compiler_params=_sc_params(),
    )
    return kern(t, src, dst)


def _stage_c_body(h_ref, w2_ref, b2_ref, sc_ref, bi_ref, g_ref):
    y = jnp.dot(h_ref[...], w2_ref[...], preferred_element_type=jnp.float32)
    y = y + b2_ref[...]
    mu = jnp.mean(y, axis=-1, keepdims=True)
    var = jnp.mean((y - mu) ** 2, axis=-1, keepdims=True)
    g_ref[...] = (y - mu) * lax.rsqrt(var + 1e-5) * sc_ref[...] + bi_ref[...]


def _stage_c(h, W2, b2, ln_scale, ln_bias):
    BE = 3200
    grid = (E // BE,)
    return pl.pallas_call(
        _stage_c_body,
        grid=grid,
        in_specs=[
            pl.BlockSpec((BE, H), lambda i: (i, 0)),
            pl.BlockSpec((H, D), lambda i: (0, 0)),
            pl.BlockSpec((1, D), lambda i: (0, 0)),
            pl.BlockSpec((1, D), lambda i: (0, 0)),
            pl.BlockSpec((1, D), lambda i: (0, 0)),
        ],
        out_specs=pl.BlockSpec((BE, D), lambda i: (i, 0)),
        out_shape=jax.ShapeDtypeStruct((E, D), jnp.float32),
    )(h, W2, b2.reshape(1, D), ln_scale.reshape(1, D), ln_bias.reshape(1, D))


DRAIN = 256


def _stage_d_kernel(g_hbm, src_hbm, out_hbm,
                    sv, sv2, mat, locv, rows0, rows1,
                    tsum, tmax,
                    cnt_smem, ptr_smem, sem0, sem1, sem_v0, sem_v1):
    c = lax.axis_index("c")
    s = lax.axis_index("s")
    wid = c * NS + s
    lo = wid * NPW

    zero16 = jnp.zeros((16,), jnp.float32)
    neg16 = jnp.full((16,), _NEG, jnp.float32)
    lane = lax.iota(jnp.int32, 16)
    true16 = lane < 16

    @pl.loop(0, TBL)
    def _init(r):
        @pl.loop(0, D, step=16)
        def _initc(t):
            tsum[r, pl.ds(t, 16)] = zero16

    @pl.loop(0, TBL)
    def _initcnt(r):
        cnt_smem[r] = 0

    @pl.loop(0, TBL)
    def _initm(r):
        @pl.loop(0, D, step=16)
        def _initmc(t):
            tmax[r, pl.ds(t, 16)] = neg16

    ptr_smem[0] = 0

    def _gather(k, rows, sem):
        return pltpu.async_copy(
            g_hbm.at[mat.at[pl.ds(k * GD, GD)]], rows, sem)

    def _consume(k, rows):
        @pl.loop(0, GD)
        def _edge(i):
            lv = locv[pl.ds(k * GD + (i // 16) * 16, 16)]
            l = jnp.max(jnp.where(lane == i % 16, lv, -1))
            cnt_smem[l] = cnt_smem[l] + 1

            @pl.loop(0, D, step=16)
            def _feat(t):
                v = rows[i, pl.ds(t, 16)]
                plsc.addupdate(tsum.at[l, pl.ds(t, 16)], v)
                tmax[l, pl.ds(t, 16)] = jnp.maximum(tmax[l, pl.ds(t, 16)], v)

    def _process_groups(ngroups):
        @pl.when(ngroups > 0)
        def _():
            _gather(0, rows0, sem0)

        @pl.loop(0, ngroups)
        def _grp(k):
            @pl.when(k % 2 == 0)
            def _():
                pltpu.make_async_copy(
                    g_hbm.at[mat.at[pl.ds(k * GD, GD)]], rows0, sem0).wait()

                @pl.when(k + 1 < ngroups)
                def _():
                    _gather(k + 1, rows1, sem1)
                _consume(k, rows0)

            @pl.when(k % 2 == 1)
            def _():
                pltpu.make_async_copy(
                    g_hbm.at[mat.at[pl.ds(k * GD, GD)]], rows1, sem1).wait()

                @pl.when(k + 1 < ngroups)
                def _():
                    _gather(k + 1, rows0, sem0)
                _consume(k, rows1)

    def _drain():
        cnt = ptr_smem[0]
        g0 = cnt // GD
        _process_groups(g0)
        rem0 = g0 * GD
        for q in range(GD // 16):
            mat[pl.ds(q * 16, 16)] = mat[pl.ds(rem0 + q * 16, 16)]
            locv[pl.ds(q * 16, 16)] = locv[pl.ds(rem0 + q * 16, 16)]
        ptr_smem[0] = cnt - rem0

    NCHD = E // CD

    def _chunk_phase(kc, svb, semv, osvb, osemv):
        pltpu.make_async_copy(
            src_hbm.at[pl.ds(kc * CD, CD)], svb, semv).wait()

        @pl.when(kc + 1 < NCHD)
        def _():
            pltpu.async_copy(
                src_hbm.at[pl.ds((kc + 1) * CD, CD)], osvb, osemv)

        @pl.loop(0, CD // 16)
        def _vec(j):
            sq = svb[pl.ds(j * 16, 16)]
            m = (sq >= lo) & (sq < lo + NPW)
            eid = kc * CD + j * 16 + lane
            cnt = ptr_smem[0]
            plsc.store_compressed(mat.at[pl.ds(cnt, 16)], eid, mask=m)
            plsc.store_compressed(locv.at[pl.ds(cnt, 16)], sq - lo, mask=m)
            ptr_smem[0] = cnt + jnp.sum(m.astype(jnp.int32))

        @pl.when(ptr_smem[0] >= DRAIN)
        def _():
            _drain()

    pltpu.async_copy(src_hbm.at[pl.ds(0, CD)], sv, sem_v0)

    @pl.loop(0, NCHD)
    def _chunk(kc):
        @pl.when(kc % 2 == 0)
        def _():
            _chunk_phase(kc, sv, sem_v0, sv2, sem_v1)

        @pl.when(kc % 2 == 1)
        def _():
            _chunk_phase(kc, sv2, sem_v1, sv, sem_v0)

    cnt = ptr_smem[0]
    for q in range(GD // 16):
        plsc.store_compressed(mat.at[pl.ds(cnt + q * 16, 16)],
                              jnp.zeros((16,), jnp.int32), mask=true16)
        plsc.store_compressed(locv.at[pl.ds(cnt + q * 16, 16)],
                              jnp.full((16,), PAD_LOC, jnp.int32), mask=true16)
    _process_groups((cnt + GD - 1) // GD)

    @pl.loop(0, NPW)
    def _node(r):
        ci = cnt_smem[r]
        cf = jnp.broadcast_to(ci, (16,)).astype(jnp.float32)
        inv = 1.0 / jnp.maximum(cf, 1.0)
        nz = cf > 0.0

        @pl.loop(0, D, step=16)
        def _feat(t):
            mean = tsum[r, pl.ds(t, 16)] * inv
            mx = jnp.where(nz, tmax[r, pl.ds(t, 16)], 0.0)
            tsum[r, pl.ds(t, 16)] = mean + mx

    pltpu.sync_copy(tsum.at[pl.ds(0, NPW)], out_hbm.at[pl.ds(lo, NPW)])


def _stage_d(g, src):
    mesh = plsc.VectorSubcoreMesh(core_axis_name="c", subcore_axis_name="s")
    kern = pl.kernel(
        _stage_d_kernel,
        out_type=jax.ShapeDtypeStruct((NPAD, D), jnp.float32),
        mesh=mesh,
        scratch_types=[
            pltpu.VMEM((CD,), jnp.int32),
            pltpu.VMEM((CD,), jnp.int32),
            pltpu.VMEM((MATBUF,), jnp.int32),
            pltpu.VMEM((MATBUF,), jnp.int32),
            pltpu.VMEM((GD, D), jnp.float32),
            pltpu.VMEM((GD, D), jnp.float32),
            pltpu.VMEM((TBL, D), jnp.float32),
            pltpu.VMEM((TBL, D), jnp.float32),
            pltpu.SMEM((TBL,), jnp.int32),
            pltpu.SMEM((1,), jnp.int32),
            pltpu.SemaphoreType.DMA,
            pltpu.SemaphoreType.DMA,
            pltpu.SemaphoreType.DMA,
            pltpu.SemaphoreType.DMA,
        ],
        compiler_params=_sc_params(),
    )
    return kern(g, src)


@functools.partial(jax.jit)
def kernel(x, edge_index, W1, b1, W2, b2, ln_scale, ln_bias):
    src = edge_index[0].astype(jnp.int32)
    dst = edge_index[1].astype(jnp.int32)
    t = _stage_a(x, W1, b1)
    h = _stage_b(t, src, dst)
    g = _stage_c(h, W2, b2, ln_scale, ln_bias)
    out = _stage_d(g, src)
    return out[:N]
